# baseline (speedup 1.0000x reference)
import functools

import jax
import jax.numpy as jnp
from jax import lax
from jax.experimental import pallas as pl
from jax.experimental.pallas import tpu as pltpu
from jax.experimental.pallas import tpu_sc as plsc

NC, NS, NW, L = 2, 16, 32, 16

f32 = jnp.float32
i32 = jnp.int32


def _mesh():
    return plsc.VectorSubcoreMesh(
        core_axis_name="c", subcore_axis_name="s", num_cores=NC, num_subcores=NS
    )


_SC_PARAMS = pltpu.CompilerParams(
    needs_layout_passes=False, use_tc_tiling_on_sc=False
)


def _iota16():
    return lax.iota(i32, 16)


def _splat(v, dtype=f32):
    return jnp.full((16,), v, dtype=dtype)


def _make_deg_kernel(NPAD, EPAD):
    ept = EPAD // NS
    nchunks = ept // 128
    rpt = NPAD // NS

    @functools.partial(
        pl.kernel,
        out_type=jax.ShapeDtypeStruct((NPAD, 8), f32),
        mesh=_mesh(),
        compiler_params=_SC_PARAMS,
        scratch_types=[
            pltpu.MemorySpace.VMEM_SHARED((NPAD, 8), f32),
            pltpu.VMEM((16, 8), f32),
            pltpu.VMEM((128, 8), f32),
            pltpu.VMEM((128,), i32),
            pltpu.SemaphoreType.DMA,
        ],
    )
    def deg_kernel(dst_hbm, ones_hbm, deg_hbm, acc_sp, zrows, ones_v, idx_v,
                   sem):
        c = lax.axis_index("c")
        s = lax.axis_index("s")
        pltpu.sync_copy(ones_hbm.at[pl.ds(0, 128)], ones_v)
        pltpu.sync_copy(ones_hbm.at[pl.ds(128, 16)], zrows)

        def zero_step(i, _):
            pltpu.sync_copy(zrows, acc_sp.at[pl.ds(s * rpt + i * 16, 16)])
            return 0

        lax.fori_loop(0, rpt // 16, zero_step, 0)
        plsc.subcore_barrier()

        def edge_step(j, _):
            pltpu.sync_copy(dst_hbm.at[pl.ds(s * ept + j * 128, 128)], idx_v)
            pltpu.sync_copy(ones_v, acc_sp.at[idx_v], add=True)
            return 0

        lax.fori_loop(0, nchunks, edge_step, 0)
        plsc.subcore_barrier()

        @pl.when(c == 0)
        def _():
            pltpu.sync_copy(
                acc_sp.at[pl.ds(s * rpt, rpt)], deg_hbm.at[pl.ds(s * rpt, rpt)]
            )

    return deg_kernel


def _make_winner_kernel(NPAD, EPAD, S):
    ept = EPAD // NW
    nchunks = ept // 128
    ngroups = S // 16

    @functools.partial(
        pl.kernel,
        out_type=jax.ShapeDtypeStruct((NW, NPAD), i32),
        mesh=_mesh(),
        compiler_params=_SC_PARAMS,
        scratch_types=[
            pltpu.MemorySpace.VMEM_SHARED((NPAD,), i32),
            pltpu.VMEM((NPAD,), i32),
            pltpu.VMEM((NPAD,), i32),
            pltpu.VMEM((S,), i32),
            pltpu.VMEM((S,), i32),
            pltpu.VMEM((32,), i32),
            pltpu.VMEM((128,), i32),
            pltpu.VMEM((128,), i32),
            pltpu.SemaphoreType.DMA,
        ],
    )
    def winner_kernel(asrc_hbm, adst_hbm, sorg_hbm, sgen_hbm, wscr_hbm,
                      map_sp, map_v, win_v, sorg_v, sgen_v, sbuf, isrc, idst,
                      sem):
        c = lax.axis_index("c")
        s = lax.axis_index("s")
        gid = c * NS + s
        iota = _iota16()

        @pl.when(s == 0)
        def _():
            def init(i, _):
                map_v[pl.ds(i * 16, 16)] = _splat(-1, i32)
                return 0

            lax.fori_loop(0, NPAD // 16, init, 0)
            pltpu.sync_copy(sorg_hbm, sorg_v)
            pltpu.sync_copy(sgen_hbm, sgen_v)
            sbuf[pl.ds(0, 16)] = _splat(-2, i32)
            sbuf[pl.ds(16, 16)] = _splat(-2, i32)
            for g in range(ngroups):
                so = sorg_v[pl.ds(g * 16, 16)]
                sg = sgen_v[pl.ds(g * 16, 16)]
                key = so * 16 + iota
                sk, sv = plsc.sort_key_val(key, sg)
                sd = lax.shift_right_logical(sk, 4)
                sbuf[pl.ds(0, 16)] = sd
                nxt = plsc.load_gather(sbuf, [iota + 1])
                plsc.store_scatter(map_v, [sd], sv, mask=sd != nxt)
            pltpu.sync_copy(map_v, map_sp)

        plsc.subcore_barrier()

        @pl.when(s != 0)
        def _():
            pltpu.sync_copy(map_sp, map_v)

        sbuf[pl.ds(16, 16)] = _splat(-2, i32)

        def initw(i, _):
            win_v[pl.ds(i * 16, 16)] = _splat(-1, i32)
            return 0

        lax.fori_loop(0, NPAD // 16, initw, 0)

        def edge_step(j, _):
            base = gid * ept + j * 128
            pltpu.sync_copy(asrc_hbm.at[pl.ds(base, 128)], isrc)
            pltpu.sync_copy(adst_hbm.at[pl.ds(base, 128)], idst)
            for g in range(8):
                sv_ = isrc[pl.ds(g * 16, 16)]
                dv_ = idst[pl.ds(g * 16, 16)]
                smap = plsc.load_gather(map_v, [sv_])
                valid = smap >= 0
                key = jnp.where(valid, dv_, NPAD + iota) * 16 + iota
                sk, sval = plsc.sort_key_val(key, smap)
                sd = lax.shift_right_logical(sk, 4)
                sbuf[pl.ds(0, 16)] = sd
                nxt = plsc.load_gather(sbuf, [iota + 1])
                m = (sd != nxt) & (sd < NPAD)
                plsc.store_scatter(win_v, [sd], sval, mask=m)
            return 0

        lax.fori_loop(0, nchunks, edge_step, 0)
        pltpu.sync_copy(win_v, wscr_hbm.at[gid])

    return winner_kernel


def _make_merge_kernel(NPAD, N):
    rpt = NPAD // NW

    @functools.partial(
        pl.kernel,
        out_type=jax.ShapeDtypeStruct((NPAD,), i32),
        mesh=_mesh(),
        compiler_params=_SC_PARAMS,
        scratch_types=[
            pltpu.VMEM((NW, rpt), i32),
            pltpu.VMEM((rpt,), i32),
            pltpu.SemaphoreType.DMA,
        ],
    )
    def merge_kernel(wscr_hbm, wsrc_hbm, tbuf, res_v, sem):
        c = lax.axis_index("c")
        s = lax.axis_index("s")
        gid = c * NS + s
        base = gid * rpt
        descs = []
        for t in range(NW):
            descs.append(
                pltpu.async_copy(wscr_hbm.at[t, pl.ds(base, rpt)], tbuf.at[t], sem)
            )
        for d in descs:
            d.wait()
        iota = _iota16()
        for r in range(rpt // 16):
            res = N + (r % 8) * 16 + iota
            for t in range(NW):
                wt = tbuf[t, pl.ds(r * 16, 16)]
                res = jnp.where(wt >= 0, wt, res)
            res_v[pl.ds(r * 16, 16)] = res
        pltpu.sync_copy(res_v, wsrc_hbm.at[pl.ds(base, rpt)])

    return merge_kernel


def _make_gather_kernel(NPAD, D):
    rpt = NPAD // NW
    nsub = rpt // 64

    @functools.partial(
        pl.kernel,
        out_type=jax.ShapeDtypeStruct((NPAD, D), f32),
        mesh=_mesh(),
        compiler_params=_SC_PARAMS,
        scratch_types=[
            pltpu.VMEM((nsub, 64), i32),
            pltpu.VMEM((rpt, D), f32),
            pltpu.SemaphoreType.DMA,
        ],
    )
    def gather_kernel(h1_hbm, wsrc_hbm, x1_hbm, idx2, rows, sem):
        c = lax.axis_index("c")
        s = lax.axis_index("s")
        gid = c * NS + s
        base = gid * rpt
        for j in range(nsub):
            pltpu.sync_copy(wsrc_hbm.at[pl.ds(base + j * 64, 64)], idx2.at[j])
        descs = []
        for j in range(nsub):
            descs.append(
                pltpu.async_copy(
                    h1_hbm.at[idx2.at[j]], rows.at[pl.ds(j * 64, 64)], sem
                )
            )
        for d in descs:
            d.wait()
        pltpu.sync_copy(rows, x1_hbm.at[pl.ds(base, rpt)])

    return gather_kernel


def _make_segsum_kernel(NPAD, EPAD, FB):
    ept = EPAD // NS
    nchunks = ept // 128
    rpt = NPAD // NS

    NB = 8
    AH = 4
    ZR = 40
    NP = 2

    @functools.partial(
        pl.kernel,
        out_type=jax.ShapeDtypeStruct((NC * NP * NPAD, FB), f32),
        mesh=_mesh(),
        compiler_params=_SC_PARAMS,
        scratch_types=[
            pltpu.MemorySpace.VMEM_SHARED((NPAD, FB), f32),
            pltpu.VMEM((ZR, FB), f32),
            pltpu.VMEM((nchunks, 128), i32),
            pltpu.VMEM((nchunks, 128), i32),
            pltpu.VMEM((NB, 128, FB), f32),
            pltpu.SemaphoreType.DMA,
            pltpu.SemaphoreType.DMA,
            pltpu.SemaphoreType.DMA,
        ],
    )
    def segsum_kernel(hs4_hbm, srclo_hbm, srchi_hbm, dst_hbm, out_hbm,
                      acc_sp, zrows, isrc, idst, rows, sem_g, sem_s, sem_z):
        c = lax.axis_index("c")
        s = lax.axis_index("s")

        @pl.when(c == 0)
        def _():
            pltpu.sync_copy(srclo_hbm.at[pl.ds(s * nchunks, nchunks)], isrc)

        @pl.when(c == 1)
        def _():
            pltpu.sync_copy(srchi_hbm.at[pl.ds(s * nchunks, nchunks)], isrc)

        pltpu.sync_copy(dst_hbm.at[pl.ds(s * nchunks, nchunks)], idst)

        def gather(j, b):
            return pltpu.async_copy(hs4_hbm.at[isrc.at[j]], rows.at[b], sem_g)

        def fillz(i, _):
            for k in range(FB // 16):
                zrows[i, pl.ds(k * 16, 16)] = _splat(0.0)
            return 0

        lax.fori_loop(0, ZR, fillz, 0)

        for p in range(NP):
            if p > 0:
                def shift(j, _):
                    for k in range(8):
                        isrc[j, pl.ds(k * 16, 16)] = (
                            isrc[j, pl.ds(k * 16, 16)] + NPAD
                        )
                    return 0

                lax.fori_loop(0, nchunks, shift, 0)
            for a in range(AH):
                gather(a, a)
            for z in range(rpt // ZR):
                pltpu.async_copy(
                    zrows, acc_sp.at[pl.ds(s * rpt + z * ZR, ZR)], sem_z
                )
            for z in range(rpt // ZR):
                pltpu.make_async_copy(
                    zrows, acc_sp.at[pl.ds(s * rpt + z * ZR, ZR)], sem_z
                ).wait()
            plsc.subcore_barrier()

            def ring_step(j0, _):
                for b in range(NB):
                    j = j0 * NB + b
                    bw = (b + AH) % NB

                    @pl.when(j >= AH)
                    def _():
                        pltpu.make_async_copy(
                            rows.at[bw], acc_sp.at[idst.at[j - AH]], sem_s
                        ).wait()

                    @pl.when(j + AH < nchunks)
                    def _():
                        gather(j + AH, bw)

                    pltpu.make_async_copy(
                        hs4_hbm.at[isrc.at[j]], rows.at[b], sem_g
                    ).wait()
                    pltpu.async_copy(
                        rows.at[b], acc_sp.at[idst.at[j]], sem_s, add=True
                    )
                return 0

            lax.fori_loop(0, nchunks // NB, ring_step, 0)
            for j in range(nchunks - AH, nchunks):
                pltpu.make_async_copy(
                    rows.at[j % NB], acc_sp.at[idst.at[j]], sem_s
                ).wait()
            plsc.subcore_barrier()
            pltpu.sync_copy(
                acc_sp.at[pl.ds(s * rpt, rpt)],
                out_hbm.at[pl.ds((c * NP + p) * NPAD + s * rpt, rpt)],
            )

    return segsum_kernel


def _elu(v):
    return jnp.where(v > 0, v, jnp.exp(jnp.minimum(v, 0.0)) - 1.0)


def _mm_scale(x, w, deg16, BM=1024):
    NPAD, K = x.shape
    NO = w.shape[1]
    FQ = NO // 4

    def body(x_ref, w_ref, deg_ref, o_ref, dv_ref):
        h = jnp.dot(x_ref[...], w_ref[...], preferred_element_type=f32)
        dv = lax.rsqrt(jnp.maximum(deg_ref[...][:, :1] + 1.0, 1e-12))
        dv_ref[...] = dv
        for q in range(4):
            o_ref[q // 2, q % 2] = h[:, q * FQ:(q + 1) * FQ] * dv

    return pl.pallas_call(
        body,
        grid=(NPAD // BM,),
        in_specs=[
            pl.BlockSpec((BM, K), lambda i: (i, 0)),
            pl.BlockSpec((K, NO), lambda i: (0, 0)),
            pl.BlockSpec((BM, 8), lambda i: (i, 0)),
        ],
        out_specs=[
            pl.BlockSpec((2, 2, BM, FQ), lambda i: (0, 0, i, 0)),
            pl.BlockSpec((BM, 1), lambda i: (i, 0)),
        ],
        out_shape=[
            jax.ShapeDtypeStruct((2, 2, NPAD, FQ), f32),
            jax.ShapeDtypeStruct((NPAD, 1), f32),
        ],
    )(x, w, deg16)


def _conv_post(S, hs, dinv, b, N, BM=1024):
    _, _, NPAD, FQ = S.shape

    def body(s_ref, hs_ref, dv_ref, b_ref, o_ref):
        i = pl.program_id(0)
        h = jnp.concatenate(
            [s_ref[q // 2, q % 2] + hs_ref[q // 2, q % 2] for q in range(4)],
            axis=1,
        )
        h = h * dv_ref[...] + b_ref[...]
        rows = i * BM + lax.broadcasted_iota(i32, (BM, 1), 0)
        o_ref[...] = jnp.where(rows < N, h, 0.0)

    return pl.pallas_call(
        body,
        grid=(NPAD // BM,),
        in_specs=[
            pl.BlockSpec((2, 2, BM, FQ), lambda i: (0, 0, i, 0)),
            pl.BlockSpec((2, 2, BM, FQ), lambda i: (0, 0, i, 0)),
            pl.BlockSpec((BM, 1), lambda i: (i, 0)),
            pl.BlockSpec((1, 4 * FQ), lambda i: (0, 0)),
        ],
        out_specs=pl.BlockSpec((BM, 4 * FQ), lambda i: (i, 0)),
        out_shape=jax.ShapeDtypeStruct((NPAD, 4 * FQ), f32),
    )(S, hs, dinv, b)


def _elu_cat_mm_scale(h1, x1, w2, dinv, BM=1024):
    NPAD, H = h1.shape
    NO = w2.shape[1]
    FQ = NO // 4

    def body(a_ref, b_ref, w_ref, dv_ref, o_ref):
        a = _elu(a_ref[...])
        b = _elu(b_ref[...])
        g = jnp.dot(a, w_ref[: H, :], preferred_element_type=f32)
        g = g + jnp.dot(b, w_ref[H:, :], preferred_element_type=f32)
        dv = dv_ref[...]
        for q in range(4):
            o_ref[q // 2, q % 2] = g[:, q * FQ:(q + 1) * FQ] * dv

    return pl.pallas_call(
        body,
        grid=(NPAD // BM,),
        in_specs=[
            pl.BlockSpec((BM, H), lambda i: (i, 0)),
            pl.BlockSpec((BM, H), lambda i: (i, 0)),
            pl.BlockSpec((2 * H, NO), lambda i: (0, 0)),
            pl.BlockSpec((BM, 1), lambda i: (i, 0)),
        ],
        out_specs=pl.BlockSpec((2, 2, BM, FQ), lambda i: (0, 0, i, 0)),
        out_shape=jax.ShapeDtypeStruct((2, 2, NPAD, FQ), f32),
    )(h1, x1, w2, dinv)


def _head(S2, gs, dinv, b2, wf, bf, maskf, BM=1024):
    _, _, NPAD, FQ = S2.shape

    def body(s_ref, g_ref, dv_ref, b_ref, wf_ref, bf_ref, m_ref, o_ref):
        h = jnp.concatenate(
            [s_ref[q // 2, q % 2] + g_ref[q // 2, q % 2] for q in range(4)],
            axis=1,
        )
        h = _elu(h * dv_ref[...] + b_ref[...])
        e = jnp.dot(h, wf_ref[...], preferred_element_type=f32) + bf_ref[...]
        o_ref[...] = jnp.where(m_ref[...] > 0, -1e9, e)

    e = pl.pallas_call(
        body,
        grid=(NPAD // BM,),
        in_specs=[
            pl.BlockSpec((2, 2, BM, FQ), lambda i: (0, 0, i, 0)),
            pl.BlockSpec((2, 2, BM, FQ), lambda i: (0, 0, i, 0)),
            pl.BlockSpec((BM, 1), lambda i: (i, 0)),
            pl.BlockSpec((1, 4 * FQ), lambda i: (0, 0)),
            pl.BlockSpec((4 * FQ, 1), lambda i: (0, 0)),
            pl.BlockSpec((1, 1), lambda i: (0, 0)),
            pl.BlockSpec((BM, 1), lambda i: (i, 0)),
        ],
        out_specs=pl.BlockSpec((BM, 1), lambda i: (i, 0)),
        out_shape=jax.ShapeDtypeStruct((NPAD, 1), f32),
    )(S2, gs, dinv, b2, wf, bf, maskf)

    def lsm_body(e_ref, o_ref):
        ev = e_ref[...]
        mx = jnp.max(ev)
        lse = jnp.log(jnp.sum(jnp.exp(ev - mx))) + mx
        o_ref[...] = ev - lse

    vm = pl.BlockSpec(memory_space=pltpu.MemorySpace.VMEM)
    return pl.pallas_call(
        lsm_body,
        in_specs=[vm],
        out_specs=vm,
        out_shape=jax.ShapeDtypeStruct((NPAD, 1), f32),
    )(e)


def kernel(x, edge_index, all_edge_index, s_mapping_index, e_mask,
           W1, b1, W2, b2, Wf, bf):
    N, D = x.shape
    E = edge_index.shape[1]
    S = s_mapping_index.shape[1]
    H = W1.shape[1]
    O = W2.shape[1]

    NPAD = ((N + NW * 16 - 1) // (NW * 16)) * (NW * 16)
    EPAD = ((E + NW * 128 - 1) // (NW * 128)) * (NW * 128)

    src = jnp.concatenate(
        [edge_index[0].astype(i32), jnp.full((EPAD - E,), N, i32)]
    )
    dst = jnp.concatenate(
        [edge_index[1].astype(i32), jnp.full((EPAD - E,), NPAD - 1, i32)]
    )
    asrc = jnp.concatenate(
        [all_edge_index[0].astype(i32), jnp.full((EPAD - E,), N, i32)]
    )
    adst = jnp.concatenate(
        [all_edge_index[1].astype(i32), jnp.full((EPAD - E,), NPAD - 1, i32)]
    )
    sgen = s_mapping_index[0].astype(i32)
    sorg = s_mapping_index[1].astype(i32)

    x_pad = jnp.pad(x, ((0, NPAD - N), (0, 0)))
    maskf = jnp.pad(e_mask.astype(f32), ((0, NPAD - N), (0, 0)),
                    constant_values=1.0)

    onesz = jnp.concatenate(
        [jnp.ones((128, 8), f32), jnp.zeros((16, 8), f32)]
    )
    deg16 = _make_deg_kernel(NPAD, EPAD)(dst, onesz)
    wscr = _make_winner_kernel(NPAD, EPAD, S)(asrc, adst, sorg, sgen)
    wsrc = _make_merge_kernel(NPAD, N)(wscr)

    segsum = _make_segsum_kernel(NPAD, EPAD, H // 4)
    srclo2 = src.reshape(EPAD // 128, 128)
    srchi2 = (src + 2 * NPAD).reshape(EPAD // 128, 128)
    dst2 = dst.reshape(EPAD // 128, 128)

    hs, dinv2 = _mm_scale(x_pad, W1, deg16)
    S1 = segsum(hs.reshape(4 * NPAD, H // 4), srclo2, srchi2, dst2)
    S1 = S1.reshape(2, 2, NPAD, H // 4)
    h1 = _conv_post(S1, hs, dinv2, b1.reshape(1, H), N)

    x1 = _make_gather_kernel(NPAD, H)(h1, wsrc)

    gs = _elu_cat_mm_scale(h1, x1, W2, dinv2)
    S2 = segsum(gs.reshape(4 * NPAD, O // 4), srclo2, srchi2, dst2)
    S2 = S2.reshape(2, 2, NPAD, O // 4)

    e_prob = _head(S2, gs, dinv2, b2.reshape(1, O), Wf, bf.reshape(1, 1),
                   maskf)
    return e_prob[:N]


# speedup vs baseline: 1.0572x; 1.0572x over previous
import functools

import jax
import jax.numpy as jnp
from jax import lax
from jax.experimental import pallas as pl
from jax.experimental.pallas import tpu as pltpu
from jax.experimental.pallas import tpu_sc as plsc

NC, NS, NW, L = 2, 16, 32, 16

f32 = jnp.float32
i32 = jnp.int32


def _mesh():
    return plsc.VectorSubcoreMesh(
        core_axis_name="c", subcore_axis_name="s", num_cores=NC, num_subcores=NS
    )


_SC_PARAMS = pltpu.CompilerParams(
    needs_layout_passes=False, use_tc_tiling_on_sc=False
)


def _iota16():
    return lax.iota(i32, 16)


def _splat(v, dtype=f32):
    return jnp.full((16,), v, dtype=dtype)


def _make_prep_kernel(NPAD, EPAD, S):
    ept = EPAD // NS
    nchunks = ept // 128
    eptw = EPAD // NW
    nchw = eptw // 128
    rpt = NPAD // NS
    ngroups = S // 16
    DR = 8

    @functools.partial(
        pl.kernel,
        out_type=(
            jax.ShapeDtypeStruct((NPAD, 8), f32),
            jax.ShapeDtypeStruct((NW, NPAD), i32),
        ),
        mesh=_mesh(),
        compiler_params=_SC_PARAMS,
        scratch_types=[
            pltpu.MemorySpace.VMEM_SHARED((NPAD, 8), f32),
            pltpu.MemorySpace.VMEM_SHARED((NPAD,), i32),
            pltpu.VMEM((nchunks, 128), i32),
            pltpu.VMEM((128, 8), f32),
            pltpu.VMEM((64, 8), f32),
            pltpu.VMEM((NPAD,), i32),
            pltpu.VMEM((NPAD,), i32),
            pltpu.VMEM((S,), i32),
            pltpu.VMEM((S,), i32),
            pltpu.VMEM((32,), i32),
            pltpu.VMEM((128,), i32),
            pltpu.VMEM((128,), i32),
            pltpu.SemaphoreType.DMA,
            pltpu.SemaphoreType.DMA,
        ],
    )
    def prep_kernel(dst2_hbm, asrc_hbm, adst_hbm, sorg_hbm, sgen_hbm,
                    ones_hbm, deg_hbm, wscr_hbm, acc_sp, map_sp, ddst,
                    ones_v, zrows, map_v, win_v, sorg_v, sgen_v, sbuf,
                    isrc, idst, sem_d, sem_z):
        c = lax.axis_index("c")
        s = lax.axis_index("s")
        gid = c * NS + s
        iota = _iota16()

        pltpu.sync_copy(ones_hbm.at[pl.ds(0, 128)], ones_v)
        pltpu.sync_copy(ones_hbm.at[pl.ds(128, 64)], zrows)
        pltpu.sync_copy(dst2_hbm.at[pl.ds(s * nchunks, nchunks)], ddst)
        for z in range(rpt // 64):
            pltpu.async_copy(
                zrows, acc_sp.at[pl.ds(s * rpt + z * 64, 64)], sem_z
            )

        @pl.when(s == 0)
        def _():
            def init(i, _):
                map_v[pl.ds(i * 16, 16)] = _splat(-1, i32)
                return 0

            lax.fori_loop(0, NPAD // 16, init, 0)
            pltpu.sync_copy(sorg_hbm, sorg_v)
            pltpu.sync_copy(sgen_hbm, sgen_v)
            sbuf[pl.ds(0, 16)] = _splat(-2, i32)
            sbuf[pl.ds(16, 16)] = _splat(-2, i32)
            for g in range(ngroups):
                so = sorg_v[pl.ds(g * 16, 16)]
                sg = sgen_v[pl.ds(g * 16, 16)]
                key = so * 16 + iota
                sk, sv = plsc.sort_key_val(key, sg)
                sd = lax.shift_right_logical(sk, 4)
                sbuf[pl.ds(0, 16)] = sd
                nxt = plsc.load_gather(sbuf, [iota + 1])
                plsc.store_scatter(map_v, [sd], sv, mask=sd != nxt)
            pltpu.sync_copy(map_v, map_sp)

        for z in range(rpt // 64):
            pltpu.make_async_copy(
                zrows, acc_sp.at[pl.ds(s * rpt + z * 64, 64)], sem_z
            ).wait()
        plsc.subcore_barrier()

        @pl.when(s != 0)
        def _():
            pltpu.sync_copy(map_sp, map_v)

        sbuf[pl.ds(16, 16)] = _splat(-2, i32)

        def initw(i, _):
            win_v[pl.ds(i * 16, 16)] = _splat(-1, i32)
            return 0

        lax.fori_loop(0, NPAD // 16, initw, 0)

        def edge_step(j, _):
            @pl.when(j >= DR)
            def _():
                pltpu.make_async_copy(
                    ones_v, acc_sp.at[ddst.at[j - DR]], sem_d
                ).wait()

            pltpu.async_copy(ones_v, acc_sp.at[ddst.at[j]], sem_d, add=True)

            @pl.when(j < nchw)
            def _():
                base = gid * eptw + j * 128
                pltpu.sync_copy(asrc_hbm.at[pl.ds(base, 128)], isrc)
                pltpu.sync_copy(adst_hbm.at[pl.ds(base, 128)], idst)
                for g in range(8):
                    sv_ = isrc[pl.ds(g * 16, 16)]
                    dv_ = idst[pl.ds(g * 16, 16)]
                    smap = plsc.load_gather(map_v, [sv_])
                    valid = smap >= 0
                    key = jnp.where(valid, dv_, NPAD + iota) * 16 + iota
                    sk, sval = plsc.sort_key_val(key, smap)
                    sd = lax.shift_right_logical(sk, 4)
                    sbuf[pl.ds(0, 16)] = sd
                    nxt = plsc.load_gather(sbuf, [iota + 1])
                    m = (sd != nxt) & (sd < NPAD)
                    plsc.store_scatter(win_v, [sd], sval, mask=m)
            return 0

        lax.fori_loop(0, nchunks, edge_step, 0)
        for j in range(nchunks - DR, nchunks):
            pltpu.make_async_copy(
                ones_v, acc_sp.at[ddst.at[j]], sem_d
            ).wait()
        plsc.subcore_barrier()

        @pl.when(c == 0)
        def _():
            pltpu.sync_copy(
                acc_sp.at[pl.ds(s * rpt, rpt)], deg_hbm.at[pl.ds(s * rpt, rpt)]
            )

        pltpu.sync_copy(win_v, wscr_hbm.at[gid])

    return prep_kernel


def _make_gather_kernel(NPAD, D, N):
    rpt = NPAD // NW
    nsub = rpt // 64

    @functools.partial(
        pl.kernel,
        out_type=jax.ShapeDtypeStruct((NPAD, D), f32),
        mesh=_mesh(),
        compiler_params=_SC_PARAMS,
        scratch_types=[
            pltpu.VMEM((NW, rpt), i32),
            pltpu.VMEM((nsub, 64), i32),
            pltpu.VMEM((rpt, D), f32),
            pltpu.SemaphoreType.DMA,
        ],
    )
    def gather_kernel(h1_hbm, wscr_hbm, x1_hbm, tbuf, res2, rows, sem):
        c = lax.axis_index("c")
        s = lax.axis_index("s")
        gid = c * NS + s
        base = gid * rpt
        iota = _iota16()
        descs = []
        for t in range(NW):
            descs.append(
                pltpu.async_copy(wscr_hbm.at[t, pl.ds(base, rpt)], tbuf.at[t],
                                 sem)
            )
        for d in descs:
            d.wait()
        for r in range(rpt // 16):
            res = N + (r % 8) * 16 + iota
            for t in range(NW):
                wt = tbuf[t, pl.ds(r * 16, 16)]
                res = jnp.where(wt >= 0, wt, res)
            res2[r // 4, pl.ds((r % 4) * 16, 16)] = res
        descs = []
        for j in range(nsub):
            descs.append(
                pltpu.async_copy(
                    h1_hbm.at[res2.at[j]], rows.at[pl.ds(j * 64, 64)], sem
                )
            )
        for d in descs:
            d.wait()
        pltpu.sync_copy(rows, x1_hbm.at[pl.ds(base, rpt)])

    return gather_kernel


def _make_segsum_kernel(NPAD, EPAD, FB):
    ept = EPAD // NS
    nchunks = ept // 128
    rpt = NPAD // NS

    NB = 4
    AH = 2
    ZR = 40
    NP = 2

    @functools.partial(
        pl.kernel,
        out_type=jax.ShapeDtypeStruct((NC * NP * NPAD, FB), f32),
        mesh=_mesh(),
        compiler_params=_SC_PARAMS,
        scratch_types=[
            pltpu.MemorySpace.VMEM_SHARED((NPAD, FB), f32),
            pltpu.VMEM((ZR, FB), f32),
            pltpu.VMEM((nchunks, 128), i32),
            pltpu.VMEM((nchunks, 128), i32),
            pltpu.VMEM((NB, 128, FB), f32),
            pltpu.SemaphoreType.DMA,
            pltpu.SemaphoreType.DMA,
            pltpu.SemaphoreType.DMA,
        ],
    )
    def segsum_kernel(hs4_hbm, srclo_hbm, srchi_hbm, dst_hbm, out_hbm,
                      acc_sp, zrows, isrc, idst, rows, sem_g, sem_s, sem_z):
        c = lax.axis_index("c")
        s = lax.axis_index("s")

        @pl.when(c == 0)
        def _():
            pltpu.sync_copy(srclo_hbm.at[pl.ds(s * nchunks, nchunks)], isrc)

        @pl.when(c == 1)
        def _():
            pltpu.sync_copy(srchi_hbm.at[pl.ds(s * nchunks, nchunks)], isrc)

        pltpu.sync_copy(dst_hbm.at[pl.ds(s * nchunks, nchunks)], idst)

        def gather(j, b):
            return pltpu.async_copy(hs4_hbm.at[isrc.at[j]], rows.at[b], sem_g)

        def fillz(i, _):
            for k in range(FB // 16):
                zrows[i, pl.ds(k * 16, 16)] = _splat(0.0)
            return 0

        lax.fori_loop(0, ZR, fillz, 0)

        for p in range(NP):
            if p > 0:
                def shift(j, _):
                    for k in range(8):
                        isrc[j, pl.ds(k * 16, 16)] = (
                            isrc[j, pl.ds(k * 16, 16)] + NPAD
                        )
                    return 0

                lax.fori_loop(0, nchunks, shift, 0)
            for a in range(AH):
                gather(a, a)
            for z in range(rpt // ZR):
                pltpu.async_copy(
                    zrows, acc_sp.at[pl.ds(s * rpt + z * ZR, ZR)], sem_z
                )
            for z in range(rpt // ZR):
                pltpu.make_async_copy(
                    zrows, acc_sp.at[pl.ds(s * rpt + z * ZR, ZR)], sem_z
                ).wait()
            plsc.subcore_barrier()

            def ring_step(j0, _):
                for b in range(NB):
                    j = j0 * NB + b
                    bw = (b + AH) % NB

                    @pl.when(j >= AH)
                    def _():
                        pltpu.make_async_copy(
                            rows.at[bw], acc_sp.at[idst.at[j - AH]], sem_s
                        ).wait()

                    @pl.when(j + AH < nchunks)
                    def _():
                        gather(j + AH, bw)

                    pltpu.make_async_copy(
                        hs4_hbm.at[isrc.at[j]], rows.at[b], sem_g
                    ).wait()
                    pltpu.async_copy(
                        rows.at[b], acc_sp.at[idst.at[j]], sem_s, add=True
                    )
                return 0

            lax.fori_loop(0, nchunks // NB, ring_step, 0)
            for j in range(nchunks - AH, nchunks):
                pltpu.make_async_copy(
                    rows.at[j % NB], acc_sp.at[idst.at[j]], sem_s
                ).wait()
            plsc.subcore_barrier()
            pltpu.sync_copy(
                acc_sp.at[pl.ds(s * rpt, rpt)],
                out_hbm.at[pl.ds((c * NP + p) * NPAD + s * rpt, rpt)],
            )

    return segsum_kernel


def _elu(v):
    return jnp.where(v > 0, v, jnp.exp(jnp.minimum(v, 0.0)) - 1.0)


def _mm_scale(x, w, deg16, BM=1024):
    NPAD, K = x.shape
    NO = w.shape[1]
    FQ = NO // 4

    def body(x_ref, w_ref, deg_ref, o_ref, dv_ref):
        h = jnp.dot(x_ref[...], w_ref[...], preferred_element_type=f32)
        dv = lax.rsqrt(jnp.maximum(deg_ref[...][:, :1] + 1.0, 1e-12))
        dv_ref[...] = dv
        for q in range(4):
            o_ref[q // 2, q % 2] = h[:, q * FQ:(q + 1) * FQ] * dv

    return pl.pallas_call(
        body,
        grid=(NPAD // BM,),
        in_specs=[
            pl.BlockSpec((BM, K), lambda i: (i, 0)),
            pl.BlockSpec((K, NO), lambda i: (0, 0)),
            pl.BlockSpec((BM, 8), lambda i: (i, 0)),
        ],
        out_specs=[
            pl.BlockSpec((2, 2, BM, FQ), lambda i: (0, 0, i, 0)),
            pl.BlockSpec((BM, 1), lambda i: (i, 0)),
        ],
        out_shape=[
            jax.ShapeDtypeStruct((2, 2, NPAD, FQ), f32),
            jax.ShapeDtypeStruct((NPAD, 1), f32),
        ],
    )(x, w, deg16)


def _conv_post(S, hs, dinv, b, N, BM=1024):
    _, _, NPAD, FQ = S.shape

    def body(s_ref, hs_ref, dv_ref, b_ref, o_ref):
        i = pl.program_id(0)
        h = jnp.concatenate(
            [s_ref[q // 2, q % 2] + hs_ref[q // 2, q % 2] for q in range(4)],
            axis=1,
        )
        h = h * dv_ref[...] + b_ref[...]
        rows = i * BM + lax.broadcasted_iota(i32, (BM, 1), 0)
        o_ref[...] = jnp.where(rows < N, h, 0.0)

    return pl.pallas_call(
        body,
        grid=(NPAD // BM,),
        in_specs=[
            pl.BlockSpec((2, 2, BM, FQ), lambda i: (0, 0, i, 0)),
            pl.BlockSpec((2, 2, BM, FQ), lambda i: (0, 0, i, 0)),
            pl.BlockSpec((BM, 1), lambda i: (i, 0)),
            pl.BlockSpec((1, 4 * FQ), lambda i: (0, 0)),
        ],
        out_specs=pl.BlockSpec((BM, 4 * FQ), lambda i: (i, 0)),
        out_shape=jax.ShapeDtypeStruct((NPAD, 4 * FQ), f32),
    )(S, hs, dinv, b)


def _elu_cat_mm_scale(h1, x1, w2, dinv, BM=1024):
    NPAD, H = h1.shape
    NO = w2.shape[1]
    FQ = NO // 4

    def body(a_ref, b_ref, w_ref, dv_ref, o_ref):
        a = _elu(a_ref[...])
        b = _elu(b_ref[...])
        g = jnp.dot(a, w_ref[: H, :], preferred_element_type=f32)
        g = g + jnp.dot(b, w_ref[H:, :], preferred_element_type=f32)
        dv = dv_ref[...]
        for q in range(4):
            o_ref[q // 2, q % 2] = g[:, q * FQ:(q + 1) * FQ] * dv

    return pl.pallas_call(
        body,
        grid=(NPAD // BM,),
        in_specs=[
            pl.BlockSpec((BM, H), lambda i: (i, 0)),
            pl.BlockSpec((BM, H), lambda i: (i, 0)),
            pl.BlockSpec((2 * H, NO), lambda i: (0, 0)),
            pl.BlockSpec((BM, 1), lambda i: (i, 0)),
        ],
        out_specs=pl.BlockSpec((2, 2, BM, FQ), lambda i: (0, 0, i, 0)),
        out_shape=jax.ShapeDtypeStruct((2, 2, NPAD, FQ), f32),
    )(h1, x1, w2, dinv)


def _head(S2, gs, dinv, b2, wf, bf, maskf, BM=1024):
    _, _, NPAD, FQ = S2.shape

    def body(s_ref, g_ref, dv_ref, b_ref, wf_ref, bf_ref, m_ref, o_ref):
        h = jnp.concatenate(
            [s_ref[q // 2, q % 2] + g_ref[q // 2, q % 2] for q in range(4)],
            axis=1,
        )
        h = _elu(h * dv_ref[...] + b_ref[...])
        e = jnp.dot(h, wf_ref[...], preferred_element_type=f32) + bf_ref[...]
        o_ref[...] = jnp.where(m_ref[...] > 0, -1e9, e)

    e = pl.pallas_call(
        body,
        grid=(NPAD // BM,),
        in_specs=[
            pl.BlockSpec((2, 2, BM, FQ), lambda i: (0, 0, i, 0)),
            pl.BlockSpec((2, 2, BM, FQ), lambda i: (0, 0, i, 0)),
            pl.BlockSpec((BM, 1), lambda i: (i, 0)),
            pl.BlockSpec((1, 4 * FQ), lambda i: (0, 0)),
            pl.BlockSpec((4 * FQ, 1), lambda i: (0, 0)),
            pl.BlockSpec((1, 1), lambda i: (0, 0)),
            pl.BlockSpec((BM, 1), lambda i: (i, 0)),
        ],
        out_specs=pl.BlockSpec((BM, 1), lambda i: (i, 0)),
        out_shape=jax.ShapeDtypeStruct((NPAD, 1), f32),
    )(S2, gs, dinv, b2, wf, bf, maskf)

    def lsm_body(e_ref, o_ref):
        ev = e_ref[...]
        mx = jnp.max(ev)
        lse = jnp.log(jnp.sum(jnp.exp(ev - mx))) + mx
        o_ref[...] = ev - lse

    vm = pl.BlockSpec(memory_space=pltpu.MemorySpace.VMEM)
    return pl.pallas_call(
        lsm_body,
        in_specs=[vm],
        out_specs=vm,
        out_shape=jax.ShapeDtypeStruct((NPAD, 1), f32),
    )(e)


def kernel(x, edge_index, all_edge_index, s_mapping_index, e_mask,
           W1, b1, W2, b2, Wf, bf):
    N, D = x.shape
    E = edge_index.shape[1]
    S = s_mapping_index.shape[1]
    H = W1.shape[1]
    O = W2.shape[1]

    NPAD = ((N + NW * 16 - 1) // (NW * 16)) * (NW * 16)
    EPAD = ((E + NW * 128 - 1) // (NW * 128)) * (NW * 128)

    src = jnp.concatenate(
        [edge_index[0].astype(i32), jnp.full((EPAD - E,), N, i32)]
    )
    dst = jnp.concatenate(
        [edge_index[1].astype(i32), jnp.full((EPAD - E,), NPAD - 1, i32)]
    )
    asrc = jnp.concatenate(
        [all_edge_index[0].astype(i32), jnp.full((EPAD - E,), N, i32)]
    )
    adst = jnp.concatenate(
        [all_edge_index[1].astype(i32), jnp.full((EPAD - E,), NPAD - 1, i32)]
    )
    sgen = s_mapping_index[0].astype(i32)
    sorg = s_mapping_index[1].astype(i32)

    x_pad = jnp.pad(x, ((0, NPAD - N), (0, 0)))
    maskf = jnp.pad(e_mask.astype(f32), ((0, NPAD - N), (0, 0)),
                    constant_values=1.0)

    onesz = jnp.concatenate(
        [jnp.ones((128, 8), f32), jnp.zeros((64, 8), f32)]
    )
    segsum = _make_segsum_kernel(NPAD, EPAD, H // 4)
    srclo2 = src.reshape(EPAD // 128, 128)
    srchi2 = (src + 2 * NPAD).reshape(EPAD // 128, 128)
    dst2 = dst.reshape(EPAD // 128, 128)

    deg16, wscr = _make_prep_kernel(NPAD, EPAD, S)(
        dst2, asrc, adst, sorg, sgen, onesz
    )

    hs, dinv2 = _mm_scale(x_pad, W1, deg16)
    S1 = segsum(hs.reshape(4 * NPAD, H // 4), srclo2, srchi2, dst2)
    S1 = S1.reshape(2, 2, NPAD, H // 4)
    h1 = _conv_post(S1, hs, dinv2, b1.reshape(1, H), N)

    x1 = _make_gather_kernel(NPAD, H, N)(h1, wscr)

    gs = _elu_cat_mm_scale(h1, x1, W2, dinv2)
    S2 = segsum(gs.reshape(4 * NPAD, O // 4), srclo2, srchi2, dst2)
    S2 = S2.reshape(2, 2, NPAD, O // 4)

    e_prob = _head(S2, gs, dinv2, b2.reshape(1, O), Wf, bf.reshape(1, 1),
                   maskf)
    return e_prob[:N]
